# baseline (speedup 1.0000x reference)
import jax
import jax.numpy as jnp
from jax.experimental import pallas as pl
from jax.experimental.pallas import tpu as pltpu

B, L, N = 16, 96, 128
PRED = 96
D, DFF = 16, 32
E, K = 8, 2
EPS = 1e-5
T = L * N
PACK = 128 // D
RPACK = T // PACK
HPACK = PACK * DFF


def _block_diag(w):
    e, a, b = w.shape
    eye = jnp.eye(PACK, dtype=w.dtype)
    return jnp.einsum('ij,eab->eiajb', eye, w).reshape(e, PACK * a, PACK * b)


PB = 2


def _moe_forecast_kernel(eidx_ref, egate_ref,
                         xnp_ref, mean_ref, std_ref,
                         swbd_ref, w10_ref, w20_ref, w11_ref, w21_ref,
                         wg1_ref, projw_ref, out_ref):
    b = pl.program_id(0)
    f32 = jnp.float32

    def expert_ffn(inp, w1, w2):
        h = jnp.dot(inp, w1, preferred_element_type=f32)
        h = jax.nn.gelu(h)
        return jnp.dot(h, w2, preferred_element_type=f32)

    for k in range(PB):
        xnp = xnp_ref[k]

        x = jnp.dot(xnp, swbd_ref[...], preferred_element_type=f32,
                    precision=jax.lax.Precision.HIGHEST)

        bb = PB * b + k
        e0 = eidx_ref[2 * bb]
        e1 = eidx_ref[2 * bb + 1]
        g0 = egate_ref[2 * bb]
        g1 = egate_ref[2 * bb + 1]
        y0 = expert_ffn(x, w10_ref[e0], w20_ref[e0])
        y1 = expert_ffn(x, w10_ref[e1], w20_ref[e1])
        out1 = (g0 * y0 + g1 * y1) + x

        colsum = jnp.sum(out1, axis=0, keepdims=True)
        xg = colsum[:, 0:D]
        for j in range(1, PACK):
            xg = xg + colsum[:, j * D:(j + 1) * D]
        xg = xg * (1.0 / T)
        logits = jnp.dot(xg, wg1_ref[...], preferred_element_type=f32)
        ii = jax.lax.broadcasted_iota(jnp.int32, (1, E), 1)
        m1 = jnp.max(logits, axis=1, keepdims=True)
        i1 = jnp.min(jnp.where(logits == m1, ii, E), axis=1, keepdims=True)
        lmasked = jnp.where(ii == i1, -jnp.inf, logits)
        m2 = jnp.max(lmasked, axis=1, keepdims=True)
        i2 = jnp.min(jnp.where(lmasked == m2, ii, E), axis=1, keepdims=True)
        u2 = jnp.exp(m2 - m1)
        s = 1.0 + u2
        ga = 1.0 / s
        gb = u2 / s

        iota_e = jax.lax.broadcasted_iota(jnp.int32, (E, 1, 1), 0)
        sela = (iota_e == i1.reshape(1, 1, 1)).astype(f32)
        selb = (iota_e == i2.reshape(1, 1, 1)).astype(f32)
        w1a = jnp.sum(w11_ref[...] * sela, axis=0)
        w2a = jnp.sum(w21_ref[...] * sela, axis=0)
        w1b = jnp.sum(w11_ref[...] * selb, axis=0)
        w2b = jnp.sum(w21_ref[...] * selb, axis=0)
        ya = expert_ffn(out1, w1a, w2a)
        yb = expert_ffn(out1, w1b, w2b)
        out2 = (ga * ya + gb * yb) + out1

        pin = out2.reshape(N, L * D)
        p = jnp.dot(pin, projw_ref[...], preferred_element_type=f32)
        out_ref[k] = jnp.transpose(p) * std_ref[k, 0] + mean_ref[k, 0]


def kernel(x_enc, x_mark_enc, x_dec, x_mark_dec, start_w, start_b,
           w_gate_0, we1_0, be1_0, we2_0, be2_0,
           w_gate_1, we1_1, be1_1, we2_1, be2_1,
           proj_w, proj_b):
    f32 = jnp.float32
    mean = jnp.mean(x_enc, axis=1, keepdims=True)
    std = jnp.sqrt(jnp.var(x_enc, axis=1, keepdims=True) + EPS)
    xn = (x_enc - mean) / std
    out0 = xn[..., None] * start_w[0] + start_b
    xg0 = jnp.mean(out0, axis=(1, 2))
    logits0 = xg0 @ w_gate_0
    ii = jnp.arange(E, dtype=jnp.int32)[None, :]
    v1 = jnp.max(logits0, axis=1, keepdims=True)
    i1 = jnp.argmax(logits0, axis=1).astype(jnp.int32)
    masked = jnp.where(ii == i1[:, None], -jnp.inf, logits0)
    v2 = jnp.max(masked, axis=1, keepdims=True)
    i2 = jnp.argmax(masked, axis=1).astype(jnp.int32)
    u = jnp.exp(v2 - v1)[:, 0]
    ga = 1.0 / (1.0 + u)
    gb = u / (1.0 + u)
    swap = i2 < i1
    e_lo = jnp.where(swap, i2, i1)
    e_hi = jnp.where(swap, i1, i2)
    g_lo = jnp.where(swap, gb, ga)
    g_hi = jnp.where(swap, ga, gb)
    eidx = jnp.stack([e_lo, e_hi], axis=1).reshape(-1)
    egate = jnp.stack([g_lo, g_hi], axis=1).reshape(-1)

    xnp = jnp.transpose(xn, (0, 2, 1)).reshape(B, RPACK, PACK)

    swbd = _block_diag(start_w[0].reshape(1, 1, D))[0]
    w10 = _block_diag(we1_0)
    w20 = _block_diag(we2_0)
    w11 = _block_diag(we1_1)
    w21 = _block_diag(we2_1)

    grid_spec = pltpu.PrefetchScalarGridSpec(
        num_scalar_prefetch=2,
        grid=(B // PB,),
        in_specs=[
            pl.BlockSpec((PB, RPACK, PACK), lambda b, *_: (b, 0, 0)),
            pl.BlockSpec((PB, 1, N), lambda b, *_: (b, 0, 0)),
            pl.BlockSpec((PB, 1, N), lambda b, *_: (b, 0, 0)),
            pl.BlockSpec((PACK, PACK * D), lambda b, *_: (0, 0)),
            pl.BlockSpec((E, PACK * D, HPACK), lambda b, *_: (0, 0, 0)),
            pl.BlockSpec((E, HPACK, PACK * D), lambda b, *_: (0, 0, 0)),
            pl.BlockSpec((E, PACK * D, HPACK), lambda b, *_: (0, 0, 0)),
            pl.BlockSpec((E, HPACK, PACK * D), lambda b, *_: (0, 0, 0)),
            pl.BlockSpec((D, E), lambda b, *_: (0, 0)),
            pl.BlockSpec((L * D, PRED), lambda b, *_: (0, 0)),
        ],
        out_specs=pl.BlockSpec((PB, PRED, N), lambda b, *_: (b, 0, 0)),
    )
    out = pl.pallas_call(
        _moe_forecast_kernel,
        grid_spec=grid_spec,
        out_shape=jax.ShapeDtypeStruct((B, PRED, N), f32),
        compiler_params=pltpu.CompilerParams(
            dimension_semantics=("parallel",)),
    )(eidx, egate, xnp, mean, std, swbd, w10, w20, w11, w21, w_gate_1, proj_w)
    return out


# speedup vs baseline: 1.0058x; 1.0058x over previous
import jax
import jax.numpy as jnp
from jax.experimental import pallas as pl
from jax.experimental.pallas import tpu as pltpu

B, L, N = 16, 96, 128
PRED = 96
D, DFF = 16, 32
E, K = 8, 2
EPS = 1e-5
T = L * N
PACK = 128 // D
RPACK = T // PACK
HPACK = PACK * DFF


def _block_diag(w):
    e, a, b = w.shape
    eye = jnp.eye(PACK, dtype=w.dtype)
    return jnp.einsum('ij,eab->eiajb', eye, w).reshape(e, PACK * a, PACK * b)


PB = 1


def _moe_forecast_kernel(eidx_ref, egate_ref,
                         xnp_ref, mean_ref, std_ref,
                         swbd_ref, w10_ref, w20_ref, w11_ref, w21_ref,
                         wg1_ref, projw_ref, out_ref):
    b = pl.program_id(0)
    f32 = jnp.float32

    def expert_ffn(inp, w1, w2):
        h = jnp.dot(inp, w1, preferred_element_type=f32)
        h = jax.nn.gelu(h)
        return jnp.dot(h, w2, preferred_element_type=f32)

    for k in range(PB):
        xnp = xnp_ref[k]

        x = jnp.dot(xnp, swbd_ref[...], preferred_element_type=f32,
                    precision=jax.lax.Precision.HIGHEST)

        bb = PB * b + k
        e0 = eidx_ref[2 * bb]
        e1 = eidx_ref[2 * bb + 1]
        g0 = egate_ref[2 * bb]
        g1 = egate_ref[2 * bb + 1]
        y0 = expert_ffn(x, w10_ref[e0], w20_ref[e0])
        y1 = expert_ffn(x, w10_ref[e1], w20_ref[e1])
        out1 = (g0 * y0 + g1 * y1) + x

        colsum = jnp.sum(out1, axis=0, keepdims=True)
        xg = colsum[:, 0:D]
        for j in range(1, PACK):
            xg = xg + colsum[:, j * D:(j + 1) * D]
        xg = xg * (1.0 / T)
        logits = jnp.dot(xg, wg1_ref[...], preferred_element_type=f32)
        ii = jax.lax.broadcasted_iota(jnp.int32, (1, E), 1)
        m1 = jnp.max(logits, axis=1, keepdims=True)
        i1 = jnp.min(jnp.where(logits == m1, ii, E), axis=1, keepdims=True)
        lmasked = jnp.where(ii == i1, -jnp.inf, logits)
        m2 = jnp.max(lmasked, axis=1, keepdims=True)
        i2 = jnp.min(jnp.where(lmasked == m2, ii, E), axis=1, keepdims=True)
        u2 = jnp.exp(m2 - m1)
        s = 1.0 + u2
        ga = 1.0 / s
        gb = u2 / s

        iota_e = jax.lax.broadcasted_iota(jnp.int32, (E, 1, 1), 0)
        sela = (iota_e == i1.reshape(1, 1, 1)).astype(f32)
        selb = (iota_e == i2.reshape(1, 1, 1)).astype(f32)
        w1a = jnp.sum(w11_ref[...] * sela, axis=0)
        w2a = jnp.sum(w21_ref[...] * sela, axis=0)
        w1b = jnp.sum(w11_ref[...] * selb, axis=0)
        w2b = jnp.sum(w21_ref[...] * selb, axis=0)
        ya = expert_ffn(out1, w1a, w2a)
        yb = expert_ffn(out1, w1b, w2b)
        out2 = (ga * ya + gb * yb) + out1

        pin = out2.reshape(N, L * D)
        p = jnp.dot(pin, projw_ref[...], preferred_element_type=f32)
        out_ref[k] = jnp.transpose(p) * std_ref[k, 0] + mean_ref[k, 0]


def kernel(x_enc, x_mark_enc, x_dec, x_mark_dec, start_w, start_b,
           w_gate_0, we1_0, be1_0, we2_0, be2_0,
           w_gate_1, we1_1, be1_1, we2_1, be2_1,
           proj_w, proj_b):
    f32 = jnp.float32
    mean = jnp.mean(x_enc, axis=1, keepdims=True)
    std = jnp.sqrt(jnp.var(x_enc, axis=1, keepdims=True) + EPS)
    xn = (x_enc - mean) / std
    out0 = xn[..., None] * start_w[0] + start_b
    xg0 = jnp.mean(out0, axis=(1, 2))
    logits0 = xg0 @ w_gate_0
    ii = jnp.arange(E, dtype=jnp.int32)[None, :]
    v1 = jnp.max(logits0, axis=1, keepdims=True)
    i1 = jnp.argmax(logits0, axis=1).astype(jnp.int32)
    masked = jnp.where(ii == i1[:, None], -jnp.inf, logits0)
    v2 = jnp.max(masked, axis=1, keepdims=True)
    i2 = jnp.argmax(masked, axis=1).astype(jnp.int32)
    u = jnp.exp(v2 - v1)[:, 0]
    ga = 1.0 / (1.0 + u)
    gb = u / (1.0 + u)
    swap = i2 < i1
    e_lo = jnp.where(swap, i2, i1)
    e_hi = jnp.where(swap, i1, i2)
    g_lo = jnp.where(swap, gb, ga)
    g_hi = jnp.where(swap, ga, gb)
    eidx = jnp.stack([e_lo, e_hi], axis=1).reshape(-1)
    egate = jnp.stack([g_lo, g_hi], axis=1).reshape(-1)

    xnp = jnp.transpose(xn, (0, 2, 1)).reshape(B, RPACK, PACK)

    swbd = _block_diag(start_w[0].reshape(1, 1, D))[0]
    w10 = _block_diag(we1_0)
    w20 = _block_diag(we2_0)
    w11 = _block_diag(we1_1)
    w21 = _block_diag(we2_1)

    grid_spec = pltpu.PrefetchScalarGridSpec(
        num_scalar_prefetch=2,
        grid=(B // PB,),
        in_specs=[
            pl.BlockSpec((PB, RPACK, PACK), lambda b, *_: (b, 0, 0)),
            pl.BlockSpec((PB, 1, N), lambda b, *_: (b, 0, 0)),
            pl.BlockSpec((PB, 1, N), lambda b, *_: (b, 0, 0)),
            pl.BlockSpec((PACK, PACK * D), lambda b, *_: (0, 0)),
            pl.BlockSpec((E, PACK * D, HPACK), lambda b, *_: (0, 0, 0)),
            pl.BlockSpec((E, HPACK, PACK * D), lambda b, *_: (0, 0, 0)),
            pl.BlockSpec((E, PACK * D, HPACK), lambda b, *_: (0, 0, 0)),
            pl.BlockSpec((E, HPACK, PACK * D), lambda b, *_: (0, 0, 0)),
            pl.BlockSpec((D, E), lambda b, *_: (0, 0)),
            pl.BlockSpec((L * D, PRED), lambda b, *_: (0, 0)),
        ],
        out_specs=pl.BlockSpec((PB, PRED, N), lambda b, *_: (b, 0, 0)),
    )
    out = pl.pallas_call(
        _moe_forecast_kernel,
        grid_spec=grid_spec,
        out_shape=jax.ShapeDtypeStruct((B, PRED, N), f32),
        compiler_params=pltpu.CompilerParams(
            dimension_semantics=("parallel",)),
    )(eidx, egate, xnp, mean, std, swbd, w10, w20, w11, w21, w_gate_1, proj_w)
    return out
